# baseline (speedup 1.0000x reference)
import jax
import jax.numpy as jnp
from jax.experimental import pallas as pl
from jax.experimental.pallas import tpu as pltpu

T = 0.07
KP = 100
KN = 4096
BLK = 16
BISECT_ITERS = 13
CW = 512


def _tail_pieces(nbank):
    off = (nbank // CW) * CW
    pieces = []
    rem = nbank - off
    while rem >= 128:
        pieces.append((off, 128))
        off += 128
        rem -= 128
    if rem:
        pieces.append((off, rem))
    return pieces


def _rl_kernel(idx_ref, points_ref, bankT_ref, out_ref, loss_ref, bf_ref):
    i = pl.program_id(0)
    nsteps = pl.num_programs(0)
    b_total = nsteps * BLK
    nbank = out_ref.shape[1]
    n_full = nbank // CW
    tails = _tail_pieces(nbank)

    p = points_ref[...]
    p = p / jnp.sqrt(jnp.sum(p * p, axis=1, keepdims=True))
    sims = jnp.dot(p, bankT_ref[...], preferred_element_type=jnp.float32,
                   precision=jax.lax.Precision.DEFAULT)
    out_ref[...] = sims
    bf_ref[...] = sims.astype(jnp.bfloat16)

    zero = jnp.zeros((BLK, CW), jnp.float32)
    zero_bf = jnp.zeros((BLK, CW), jnp.bfloat16)

    def count_pass(mid4, mid1):
        m4b = mid4.astype(jnp.bfloat16)
        m1b = mid1.astype(jnp.bfloat16)
        def body(j, carry):
            a4, a1 = carry
            base = pl.multiple_of(j * CW, CW)
            c = bf_ref[:, pl.ds(base, CW)]
            return (a4 + (c > m4b).astype(jnp.bfloat16),
                    a1 + (c > m1b).astype(jnp.bfloat16))
        a4, a1 = jax.lax.fori_loop(0, n_full, body, (zero_bf, zero_bf), unroll=16)
        c4 = jnp.sum(a4.astype(jnp.float32), axis=1, keepdims=True)
        c1 = jnp.sum(a1.astype(jnp.float32), axis=1, keepdims=True)
        for (o, w) in tails:
            c = out_ref[:, pl.ds(o, w)]
            c4 = c4 + jnp.sum((c > mid4).astype(jnp.float32), axis=1, keepdims=True)
            c1 = c1 + jnp.sum((c > mid1).astype(jnp.float32), axis=1, keepdims=True)
        return c4, c1

    lo0 = jnp.full((BLK, 1), -1.1, jnp.float32)
    hi0 = jnp.full((BLK, 1), 1.1, jnp.float32)

    def bisect_body(_, carry):
        lo4, hi4, lo1, hi1 = carry
        mid4 = 0.5 * (lo4 + hi4)
        mid1 = 0.5 * (lo1 + hi1)
        c4, c1 = count_pass(mid4, mid1)
        g4 = c4 >= KN
        g1 = c1 >= KP
        return (jnp.where(g4, mid4, lo4), jnp.where(g4, hi4, mid4),
                jnp.where(g1, mid1, lo1), jnp.where(g1, hi1, mid1))

    lo4, hi4, lo1, hi1 = jax.lax.fori_loop(
        0, BISECT_ITERS, bisect_body, (lo0, hi0, lo0, hi0))
    t4 = 0.5 * (lo4 + hi4)
    t1 = 0.5 * (lo1 + hi1)

    def final_body(j, carry):
        a4, a1, s4, s1 = carry
        base = pl.multiple_of(j * CW, CW)
        c = out_ref[:, pl.ds(base, CW)]
        e = jnp.exp(c / T)
        m4 = c > t4
        m1 = c > t1
        return (a4 + m4.astype(jnp.float32), a1 + m1.astype(jnp.float32),
                s4 + jnp.where(m4, e, 0.0), s1 + jnp.where(m1, e, 0.0))

    a4, a1, s4, s1 = jax.lax.fori_loop(
        0, n_full, final_body, (zero, zero, zero, zero), unroll=8)
    c4 = jnp.sum(a4, axis=1, keepdims=True)
    c1 = jnp.sum(a1, axis=1, keepdims=True)
    e4 = jnp.sum(s4, axis=1, keepdims=True)
    e1 = jnp.sum(s1, axis=1, keepdims=True)
    for (o, w) in tails:
        c = out_ref[:, pl.ds(o, w)]
        e = jnp.exp(c / T)
        m4 = c > t4
        m1 = c > t1
        c4 = c4 + jnp.sum(m4.astype(jnp.float32), axis=1, keepdims=True)
        c1 = c1 + jnp.sum(m1.astype(jnp.float32), axis=1, keepdims=True)
        e4 = e4 + jnp.sum(jnp.where(m4, e, 0.0), axis=1, keepdims=True)
        e1 = e1 + jnp.sum(jnp.where(m1, e, 0.0), axis=1, keepdims=True)

    sum_top_kn = e4 + (KN - c4) * jnp.exp(t4 / T)
    sum_top_kp = e1 + (KP - c1) * jnp.exp(t1 / T)

    lane = jax.lax.broadcasted_iota(jnp.int32, (1, 128), 1)
    vals = []
    for r in range(BLK):
        idx = idx_ref[i * BLK + r]
        base = pl.multiple_of((idx // 128) * 128, 128)
        chunk = out_ref[r, pl.ds(base, 128)].reshape(1, 128)
        sel = jnp.where(lane == (idx - base), chunk, 0.0)
        vals.append(jnp.sum(sel, axis=1, keepdims=True))
    pos = jnp.exp(jnp.concatenate(vals, axis=0) / T)

    total_pos = pos + sum_top_kp
    row_terms = jnp.log(total_pos / sum_top_kn + 1e-7)
    partial = jnp.sum(row_terms, axis=0, keepdims=True) / b_total

    prev = jnp.where(i == 0, jnp.zeros((1, 1), jnp.float32), loss_ref[...])
    loss_ref[...] = prev - partial


def kernel(points, point_indices, memory_bank):
    b, d = points.shape
    nbank = memory_bank.shape[0]
    nsteps = b // BLK
    bank_t = memory_bank.T
    idx = point_indices.astype(jnp.int32)

    grid_spec = pltpu.PrefetchScalarGridSpec(
        num_scalar_prefetch=1,
        grid=(nsteps,),
        in_specs=[
            pl.BlockSpec((BLK, d), lambda i, idx: (i, 0)),
            pl.BlockSpec((d, nbank), lambda i, idx: (0, 0)),
        ],
        out_specs=[
            pl.BlockSpec((BLK, nbank), lambda i, idx: (i, 0)),
            pl.BlockSpec((1, 1), lambda i, idx: (0, 0)),
        ],
        scratch_shapes=[pltpu.VMEM((BLK, nbank), jnp.bfloat16)],
    )
    sims, loss = pl.pallas_call(
        _rl_kernel,
        grid_spec=grid_spec,
        out_shape=[
            jax.ShapeDtypeStruct((b, nbank), jnp.float32),
            jax.ShapeDtypeStruct((1, 1), jnp.float32),
        ],
    )(idx, points, bank_t)
    return (loss[0, 0], sims)


# speedup vs baseline: 1.5421x; 1.5421x over previous
import jax
import jax.numpy as jnp
from jax.experimental import pallas as pl
from jax.experimental.pallas import tpu as pltpu

T = 0.07
KP = 100
KN = 4096
BLK = 16
BISECT_ITERS = 11
CW = 512


def _tail_pieces(nbank):
    off = (nbank // CW) * CW
    pieces = []
    rem = nbank - off
    while rem >= 128:
        pieces.append((off, 128))
        off += 128
        rem -= 128
    if rem:
        pieces.append((off, rem))
    return pieces


def _rl_kernel(idx_ref, points_ref, bankT_ref, out_ref, loss_ref):
    i = pl.program_id(0)
    nsteps = pl.num_programs(0)
    b_total = nsteps * BLK
    nbank = out_ref.shape[1]
    n_full = nbank // CW
    tails = _tail_pieces(nbank)

    p = points_ref[...]
    p = p / jnp.sqrt(jnp.sum(p * p, axis=1, keepdims=True))
    out_ref[...] = jnp.dot(p, bankT_ref[...], preferred_element_type=jnp.float32,
                           precision=jax.lax.Precision.DEFAULT)

    zero = jnp.zeros((BLK, CW), jnp.float32)

    def count_pass(mid4, mid1):
        def body(j, carry):
            a4, a1 = carry
            base = pl.multiple_of(j * CW, CW)
            c = out_ref[:, pl.ds(base, CW)]
            return (a4 + (c > mid4).astype(jnp.float32),
                    a1 + (c > mid1).astype(jnp.float32))
        a4, a1 = jax.lax.fori_loop(0, n_full, body, (zero, zero), unroll=16)
        c4 = jnp.sum(a4, axis=1, keepdims=True)
        c1 = jnp.sum(a1, axis=1, keepdims=True)
        for (o, w) in tails:
            c = out_ref[:, pl.ds(o, w)]
            c4 = c4 + jnp.sum((c > mid4).astype(jnp.float32), axis=1, keepdims=True)
            c1 = c1 + jnp.sum((c > mid1).astype(jnp.float32), axis=1, keepdims=True)
        return c4, c1

    lo0 = jnp.full((BLK, 1), -1.1, jnp.float32)
    hi0 = jnp.full((BLK, 1), 1.1, jnp.float32)

    def bisect_body(_, carry):
        lo4, hi4, lo1, hi1 = carry
        mid4 = 0.5 * (lo4 + hi4)
        mid1 = 0.5 * (lo1 + hi1)
        c4, c1 = count_pass(mid4, mid1)
        g4 = c4 >= KN
        g1 = c1 >= KP
        return (jnp.where(g4, mid4, lo4), jnp.where(g4, hi4, mid4),
                jnp.where(g1, mid1, lo1), jnp.where(g1, hi1, mid1))

    lo4, hi4, lo1, hi1 = jax.lax.fori_loop(
        0, BISECT_ITERS, bisect_body, (lo0, hi0, lo0, hi0))
    t4 = 0.5 * (lo4 + hi4)
    t1 = 0.5 * (lo1 + hi1)

    def final_body(j, carry):
        a4, a1, s4, s1 = carry
        base = pl.multiple_of(j * CW, CW)
        c = out_ref[:, pl.ds(base, CW)]
        e = jnp.exp(c / T)
        m4 = c > t4
        m1 = c > t1
        return (a4 + m4.astype(jnp.float32), a1 + m1.astype(jnp.float32),
                s4 + jnp.where(m4, e, 0.0), s1 + jnp.where(m1, e, 0.0))

    a4, a1, s4, s1 = jax.lax.fori_loop(
        0, n_full, final_body, (zero, zero, zero, zero), unroll=8)
    c4 = jnp.sum(a4, axis=1, keepdims=True)
    c1 = jnp.sum(a1, axis=1, keepdims=True)
    e4 = jnp.sum(s4, axis=1, keepdims=True)
    e1 = jnp.sum(s1, axis=1, keepdims=True)
    for (o, w) in tails:
        c = out_ref[:, pl.ds(o, w)]
        e = jnp.exp(c / T)
        m4 = c > t4
        m1 = c > t1
        c4 = c4 + jnp.sum(m4.astype(jnp.float32), axis=1, keepdims=True)
        c1 = c1 + jnp.sum(m1.astype(jnp.float32), axis=1, keepdims=True)
        e4 = e4 + jnp.sum(jnp.where(m4, e, 0.0), axis=1, keepdims=True)
        e1 = e1 + jnp.sum(jnp.where(m1, e, 0.0), axis=1, keepdims=True)

    sum_top_kn = e4 + (KN - c4) * jnp.exp(t4 / T)
    sum_top_kp = e1 + (KP - c1) * jnp.exp(t1 / T)

    lane = jax.lax.broadcasted_iota(jnp.int32, (1, 128), 1)
    vals = []
    for r in range(BLK):
        idx = idx_ref[i * BLK + r]
        base = pl.multiple_of((idx // 128) * 128, 128)
        chunk = out_ref[r, pl.ds(base, 128)].reshape(1, 128)
        sel = jnp.where(lane == (idx - base), chunk, 0.0)
        vals.append(jnp.sum(sel, axis=1, keepdims=True))
    pos = jnp.exp(jnp.concatenate(vals, axis=0) / T)

    total_pos = pos + sum_top_kp
    row_terms = jnp.log(total_pos / sum_top_kn + 1e-7)
    partial = jnp.sum(row_terms, axis=0, keepdims=True) / b_total

    prev = jnp.where(i == 0, jnp.zeros((1, 1), jnp.float32), loss_ref[...])
    loss_ref[...] = prev - partial


def kernel(points, point_indices, memory_bank):
    b, d = points.shape
    nbank = memory_bank.shape[0]
    nsteps = b // BLK
    bank_t = memory_bank.T
    idx = point_indices.astype(jnp.int32)

    grid_spec = pltpu.PrefetchScalarGridSpec(
        num_scalar_prefetch=1,
        grid=(nsteps,),
        in_specs=[
            pl.BlockSpec((BLK, d), lambda i, idx: (i, 0)),
            pl.BlockSpec((d, nbank), lambda i, idx: (0, 0)),
        ],
        out_specs=[
            pl.BlockSpec((BLK, nbank), lambda i, idx: (i, 0)),
            pl.BlockSpec((1, 1), lambda i, idx: (0, 0)),
        ],
    )
    sims, loss = pl.pallas_call(
        _rl_kernel,
        grid_spec=grid_spec,
        out_shape=[
            jax.ShapeDtypeStruct((b, nbank), jnp.float32),
            jax.ShapeDtypeStruct((1, 1), jnp.float32),
        ],
    )(idx, points, bank_t)
    return (loss[0, 0], sims)
